# baseline (device time: 22515 ns/iter reference)
import os

import jax
import jax.numpy as jnp
from jax import lax
from jax.experimental import pallas as pl
from jax.experimental.pallas import tpu as pltpu

_KMODE = "full"
_flag = os.path.join(os.path.dirname(os.path.abspath(__file__)), "kmode.txt")
if os.path.exists(_flag):
    _KMODE = open(_flag).read().strip() or "full"

N_DEV = 4
B_PER = 2
SQ = 128
HQ_PER = 4
DH = 64
D_MODEL = 512
D_GRP = HQ_PER * DH
BLK = 64


def kernel(x, Wq, K_ext, V_ext, Wo):
    my = lax.axis_index("i")
    k16 = lax.dynamic_slice_in_dim(K_ext, my * B_PER, B_PER, 0).astype(
        jnp.bfloat16)
    v16 = lax.dynamic_slice_in_dim(V_ext, my * B_PER, B_PER, 0).astype(
        jnp.bfloat16)
    karr = k16.transpose(2, 0, 1, 3).reshape(N_DEV, HQ_PER, B_PER, SQ, DH)
    varr = v16.transpose(2, 0, 1, 3).reshape(N_DEV, HQ_PER, B_PER, SQ, DH)
    x16 = x.reshape(B_PER * SQ, D_MODEL).astype(jnp.bfloat16)

    wq_s = Wq * 0.125
    sq = jnp.max(jnp.abs(wq_s), axis=0) / 127.0
    wq_i8 = jnp.round(wq_s / sq[None, :]).astype(jnp.int8)
    so = jnp.max(jnp.abs(Wo), axis=0) / 127.0
    wo_i8 = jnp.round(Wo / so[None, :]).astype(jnp.int8)
    scales = jnp.concatenate([sq, so]).reshape(1, D_GRP + D_MODEL)

    def body(x_ref, wq_ref, k_ref, v_ref, wo_ref, sc_ref, out_ref,
             wq_comm, wo_comm, sc_comm,
             wq_send, wq_recv, wo_send, wo_recv, sc_send, sc_recv):
        my_pos = lax.axis_index("i")

        sends = []
        if _KMODE != "compute":
            barrier_sem = pltpu.get_barrier_semaphore()
            for d in (1, 2, 3):
                pl.semaphore_signal(
                    barrier_sem, inc=1,
                    device_id=(lax.rem(my_pos + d, N_DEV),),
                    device_id_type=pl.DeviceIdType.MESH,
                )
            pl.semaphore_wait(barrier_sem, 3)

            for d in (1, 3, 2):
                tgt = lax.rem(my_pos + d, N_DEV)
                for src, comm, ssem, rsem in (
                    (sc_ref, sc_comm, sc_send, sc_recv),
                    (wq_ref, wq_comm, wq_send, wq_recv),
                    (wo_ref, wo_comm, wo_send, wo_recv),
                ):
                    rdma = pltpu.make_async_remote_copy(
                        src_ref=src,
                        dst_ref=comm.at[my_pos],
                        send_sem=ssem.at[d - 1],
                        recv_sem=rsem.at[my_pos],
                        device_id=(tgt,),
                        device_id_type=pl.DeviceIdType.MESH,
                    )
                    rdma.start()
                    sends.append(rdma)

        ri = lax.broadcasted_iota(jnp.int32, (SQ, SQ), 0)
        ci = lax.broadcasted_iota(jnp.int32, (SQ, SQ), 1)
        mask = (ci // BLK) <= (ri // BLK)

        def compute_group(g, wq_i8, wo_i8, sc, is_first):
            kg = k_ref.at[g]
            vg = v_ref.at[g]
            q_raw = jnp.dot(x_ref[...], wq_i8.astype(jnp.bfloat16),
                            preferred_element_type=jnp.float32)
            q16 = (q_raw * sc[:, :D_GRP]).astype(jnp.bfloat16)
            head_cols = []
            for hh in range(HQ_PER):
                batch_rows = []
                for b in range(B_PER):
                    q = q16[b * SQ:(b + 1) * SQ, hh * DH:(hh + 1) * DH]
                    k = kg[hh, b]
                    v = vg[hh, b]
                    sc_ = lax.dot_general(
                        q, k, (((1,), (1,)), ((), ())),
                        preferred_element_type=jnp.float32)
                    w = jnp.exp(jnp.where(mask, sc_, -1e9))
                    denom = jnp.sum(w, axis=1, keepdims=True)
                    ctx_h = jnp.dot(w.astype(jnp.bfloat16), v,
                                    preferred_element_type=jnp.float32)
                    batch_rows.append(ctx_h / denom)
                head_cols.append(jnp.concatenate(batch_rows, axis=0))
            ctx = jnp.concatenate(head_cols, axis=1)
            contrib = jnp.dot(ctx.astype(jnp.bfloat16),
                              wo_i8.astype(jnp.bfloat16),
                              preferred_element_type=jnp.float32)
            contrib = contrib * sc[:, D_GRP:]
            for b in range(B_PER):
                piece = contrib[b * SQ:(b + 1) * SQ, :]
                if is_first:
                    out_ref[b] = piece
                else:
                    out_ref[b] = out_ref[b] + piece

        if _KMODE == "comm":
            for b in range(B_PER):
                out_ref[b] = jnp.zeros((SQ, D_MODEL), jnp.float32)
        else:
            compute_group(my_pos, wq_ref[...], wo_ref[...], sc_ref[...],
                          is_first=True)

        for d in (3, 1, 2):
            g = lax.rem(my_pos + d, N_DEV)
            if _KMODE != "compute":
                for comm, ssem, rsem in (
                    (sc_comm, sc_send, sc_recv),
                    (wq_comm, wq_send, wq_recv),
                    (wo_comm, wo_send, wo_recv),
                ):
                    recv = pltpu.make_async_remote_copy(
                        src_ref=comm.at[g],
                        dst_ref=comm.at[g],
                        send_sem=ssem.at[0],
                        recv_sem=rsem.at[g],
                        device_id=(g,),
                        device_id_type=pl.DeviceIdType.MESH,
                    )
                    recv.wait_recv()
            if _KMODE != "comm":
                if _KMODE == "compute":
                    compute_group(g, wq_ref[...], wo_ref[...], sc_ref[...],
                                  is_first=False)
                else:
                    compute_group(g, wq_comm[g], wo_comm[g], sc_comm[g],
                                  is_first=False)

        for rdma in sends:
            rdma.wait_send()

    return pl.pallas_call(
        body,
        out_shape=jax.ShapeDtypeStruct((B_PER, SQ, D_MODEL), jnp.float32),
        in_specs=[pl.BlockSpec(memory_space=pltpu.VMEM)] * 6,
        out_specs=pl.BlockSpec(memory_space=pltpu.VMEM),
        scratch_shapes=[
            pltpu.VMEM((N_DEV, D_MODEL, D_GRP), jnp.int8),
            pltpu.VMEM((N_DEV, D_GRP, D_MODEL), jnp.int8),
            pltpu.VMEM((N_DEV, 1, D_GRP + D_MODEL), jnp.float32),
            pltpu.SemaphoreType.DMA((3,)),
            pltpu.SemaphoreType.DMA((N_DEV,)),
            pltpu.SemaphoreType.DMA((3,)),
            pltpu.SemaphoreType.DMA((N_DEV,)),
            pltpu.SemaphoreType.DMA((3,)),
            pltpu.SemaphoreType.DMA((N_DEV,)),
        ],
        compiler_params=pltpu.CompilerParams(
            collective_id=None if _KMODE == "compute" else 0),
    )(x16, wq_i8, karr, varr, wo_i8, scales)


# device time: 16611 ns/iter; 1.3554x vs baseline; 1.3554x over previous
import os

import jax
import jax.numpy as jnp
from jax import lax
from jax.experimental import pallas as pl
from jax.experimental.pallas import tpu as pltpu

_KMODE = "full"
_flag = os.path.join(os.path.dirname(os.path.abspath(__file__)), "kmode.txt")
if os.path.exists(_flag):
    _KMODE = open(_flag).read().strip() or "full"

N_DEV = 4
B_PER = 2
SQ = 128
HQ_PER = 4
DH = 64
D_MODEL = 512
D_GRP = HQ_PER * DH
BLK = 64


def kernel(x, Wq, K_ext, V_ext, Wo):
    my = lax.axis_index("i")
    k16 = lax.dynamic_slice_in_dim(K_ext, my * B_PER, B_PER, 0).astype(
        jnp.bfloat16)
    v16 = lax.dynamic_slice_in_dim(V_ext, my * B_PER, B_PER, 0).astype(
        jnp.bfloat16)
    karr = k16.transpose(2, 0, 1, 3).reshape(N_DEV, HQ_PER, B_PER, SQ, DH)
    varr = v16.transpose(2, 0, 1, 3).reshape(N_DEV, HQ_PER, B_PER, SQ, DH)

    def body(x_ref, wq_ref, k_ref, v_ref, wo_ref, out_ref,
             wq_comm, wo_comm, sc_comm, wq_stage, wo_stage, sc_stage,
             wq_send, wq_recv, wo_send, wo_recv, sc_send, sc_recv):
        my_pos = lax.axis_index("i")

        sends = []
        if _KMODE != "compute":
            barrier_sem = pltpu.get_barrier_semaphore()
            for d in (1, 2, 3):
                pl.semaphore_signal(
                    barrier_sem, inc=1,
                    device_id=(lax.rem(my_pos + d, N_DEV),),
                    device_id_type=pl.DeviceIdType.MESH,
                )

        wqv = wq_ref[...] * 0.125
        sq = jnp.max(jnp.abs(wqv), axis=0, keepdims=True) / 127.0
        wq_stage[...] = jnp.round(wqv / sq).astype(jnp.int8)
        wov = wo_ref[...]
        so = jnp.max(jnp.abs(wov), axis=0, keepdims=True) / 127.0
        wo_stage[...] = jnp.round(wov / so).astype(jnp.int8)
        sc_stage[:, :D_GRP] = sq
        sc_stage[:, D_GRP:] = so

        if _KMODE != "compute":
            pl.semaphore_wait(barrier_sem, 3)

            for d in (1, 3, 2):
                tgt = lax.rem(my_pos + d, N_DEV)
                for src, comm, ssem, rsem in (
                    (sc_stage, sc_comm, sc_send, sc_recv),
                    (wq_stage, wq_comm, wq_send, wq_recv),
                    (wo_stage, wo_comm, wo_send, wo_recv),
                ):
                    rdma = pltpu.make_async_remote_copy(
                        src_ref=src,
                        dst_ref=comm.at[my_pos],
                        send_sem=ssem.at[d - 1],
                        recv_sem=rsem.at[my_pos],
                        device_id=(tgt,),
                        device_id_type=pl.DeviceIdType.MESH,
                    )
                    rdma.start()
                    sends.append(rdma)

        x16 = x_ref[...].reshape(B_PER * SQ, D_MODEL).astype(jnp.bfloat16)
        ri = lax.broadcasted_iota(jnp.int32, (SQ, SQ), 0)
        ci = lax.broadcasted_iota(jnp.int32, (SQ, SQ), 1)
        mask = (ci // BLK) <= (ri // BLK)

        def compute_group(g, wq16, wo16, q_scale, o_scale, is_first):
            kg = k_ref.at[g]
            vg = v_ref.at[g]
            q_raw = jnp.dot(x16, wq16,
                            preferred_element_type=jnp.float32)
            if q_scale is not None:
                q_raw = q_raw * q_scale
            q16 = q_raw.astype(jnp.bfloat16)
            head_cols = []
            for hh in range(HQ_PER):
                batch_rows = []
                for b in range(B_PER):
                    q = q16[b * SQ:(b + 1) * SQ, hh * DH:(hh + 1) * DH]
                    k = kg[hh, b]
                    v = vg[hh, b]
                    s = lax.dot_general(
                        q, k, (((1,), (1,)), ((), ())),
                        preferred_element_type=jnp.float32)
                    w = jnp.exp(jnp.where(mask, s, -1e9))
                    denom = jnp.sum(w, axis=1, keepdims=True)
                    ctx_h = jnp.dot(w.astype(jnp.bfloat16), v,
                                    preferred_element_type=jnp.float32)
                    batch_rows.append(ctx_h / denom)
                head_cols.append(jnp.concatenate(batch_rows, axis=0))
            ctx = jnp.concatenate(head_cols, axis=1)
            contrib = jnp.dot(ctx.astype(jnp.bfloat16), wo16,
                              preferred_element_type=jnp.float32)
            if o_scale is not None:
                contrib = contrib * o_scale
            for b in range(B_PER):
                piece = contrib[b * SQ:(b + 1) * SQ, :]
                if is_first:
                    out_ref[b] = piece
                else:
                    out_ref[b] = out_ref[b] + piece

        if _KMODE == "comm":
            for b in range(B_PER):
                out_ref[b] = jnp.zeros((SQ, D_MODEL), jnp.float32)
        else:
            compute_group(my_pos, wqv.astype(jnp.bfloat16),
                          wov.astype(jnp.bfloat16), None, None,
                          is_first=True)

        for d in (3, 1, 2):
            g = lax.rem(my_pos + d, N_DEV)
            if _KMODE != "compute":
                for comm, ssem, rsem in (
                    (sc_comm, sc_send, sc_recv),
                    (wq_comm, wq_send, wq_recv),
                    (wo_comm, wo_send, wo_recv),
                ):
                    recv = pltpu.make_async_remote_copy(
                        src_ref=comm.at[g],
                        dst_ref=comm.at[g],
                        send_sem=ssem.at[0],
                        recv_sem=rsem.at[g],
                        device_id=(g,),
                        device_id_type=pl.DeviceIdType.MESH,
                    )
                    recv.wait_recv()
            if _KMODE != "comm":
                if _KMODE == "compute":
                    compute_group(g, wqv.astype(jnp.bfloat16),
                                  wov.astype(jnp.bfloat16), None, None,
                                  is_first=False)
                else:
                    sc = sc_comm[g]
                    compute_group(g, wq_comm[g].astype(jnp.bfloat16),
                                  wo_comm[g].astype(jnp.bfloat16),
                                  sc[:, :D_GRP], sc[:, D_GRP:],
                                  is_first=False)

        for rdma in sends:
            rdma.wait_send()

    return pl.pallas_call(
        body,
        out_shape=jax.ShapeDtypeStruct((B_PER, SQ, D_MODEL), jnp.float32),
        in_specs=[pl.BlockSpec(memory_space=pltpu.VMEM)] * 5,
        out_specs=pl.BlockSpec(memory_space=pltpu.VMEM),
        scratch_shapes=[
            pltpu.VMEM((N_DEV, D_MODEL, D_GRP), jnp.int8),
            pltpu.VMEM((N_DEV, D_GRP, D_MODEL), jnp.int8),
            pltpu.VMEM((N_DEV, 1, D_GRP + D_MODEL), jnp.float32),
            pltpu.VMEM((D_MODEL, D_GRP), jnp.int8),
            pltpu.VMEM((D_GRP, D_MODEL), jnp.int8),
            pltpu.VMEM((1, D_GRP + D_MODEL), jnp.float32),
            pltpu.SemaphoreType.DMA((3,)),
            pltpu.SemaphoreType.DMA((N_DEV,)),
            pltpu.SemaphoreType.DMA((3,)),
            pltpu.SemaphoreType.DMA((N_DEV,)),
            pltpu.SemaphoreType.DMA((3,)),
            pltpu.SemaphoreType.DMA((N_DEV,)),
        ],
        compiler_params=pltpu.CompilerParams(
            collective_id=None if _KMODE == "compute" else 0),
    )(x, Wq, karr, varr, Wo)


# device time: 16050 ns/iter; 1.4028x vs baseline; 1.0350x over previous
import os

import jax
import jax.numpy as jnp
from jax import lax
from jax.experimental import pallas as pl
from jax.experimental.pallas import tpu as pltpu

_KMODE = "full"
_flag = os.path.join(os.path.dirname(os.path.abspath(__file__)), "kmode.txt")
if os.path.exists(_flag):
    _KMODE = open(_flag).read().strip() or "full"

N_DEV = 4
B_PER = 2
SQ = 128
HQ_PER = 4
DH = 64
D_MODEL = 512
D_GRP = HQ_PER * DH
BLK = 64


def kernel(x, Wq, K_ext, V_ext, Wo):
    my = lax.axis_index("i")
    k16 = lax.dynamic_slice_in_dim(K_ext, my * B_PER, B_PER, 0).astype(
        jnp.bfloat16)
    v16 = lax.dynamic_slice_in_dim(V_ext, my * B_PER, B_PER, 0).astype(
        jnp.bfloat16)
    karr = k16.transpose(2, 0, 1, 3).reshape(N_DEV, HQ_PER, B_PER, SQ, DH)
    varr = v16.transpose(2, 0, 1, 3).reshape(N_DEV, HQ_PER, B_PER, SQ, DH)

    def body(x_ref, wq_ref, k_ref, v_ref, wo_ref, out_ref,
             wq_comm, wo_comm, sc_comm, wq_stage, wo_stage, sc_stage,
             wq_send, wq_recv, wo_send, wo_recv, sc_send, sc_recv):
        my_pos = lax.axis_index("i")

        sends = []
        if _KMODE != "compute":
            barrier_sem = pltpu.get_barrier_semaphore()
            for d in (1, 2, 3):
                pl.semaphore_signal(
                    barrier_sem, inc=1,
                    device_id=(lax.rem(my_pos + d, N_DEV),),
                    device_id_type=pl.DeviceIdType.MESH,
                )

        wqv = wq_ref[...] * 0.125
        sq = jnp.max(jnp.abs(wqv), axis=0, keepdims=True) / 127.0
        wq_stage[...] = jnp.round(wqv / sq).astype(jnp.int8)
        wov = wo_ref[...]
        so = jnp.max(jnp.abs(wov), axis=0, keepdims=True) / 127.0
        wo_stage[...] = jnp.round(wov / so).astype(jnp.int8)
        sc_stage[:, :D_GRP] = sq
        sc_stage[:, D_GRP:] = so

        if _KMODE != "compute":
            pl.semaphore_wait(barrier_sem, 3)

            for d in (1, 3, 2):
                tgt = lax.rem(my_pos + d, N_DEV)
                for src, comm, ssem, rsem in (
                    (sc_stage, sc_comm, sc_send, sc_recv),
                    (wq_stage, wq_comm, wq_send, wq_recv),
                    (wo_stage, wo_comm, wo_send, wo_recv),
                ):
                    rdma = pltpu.make_async_remote_copy(
                        src_ref=src,
                        dst_ref=comm.at[my_pos],
                        send_sem=ssem.at[d - 1],
                        recv_sem=rsem.at[my_pos],
                        device_id=(tgt,),
                        device_id_type=pl.DeviceIdType.MESH,
                    )
                    rdma.start()
                    sends.append(rdma)

        x16 = x_ref[...].reshape(B_PER * SQ, D_MODEL).astype(jnp.bfloat16)
        ri = lax.broadcasted_iota(jnp.int32, (SQ, SQ), 0)
        ci = lax.broadcasted_iota(jnp.int32, (SQ, SQ), 1)
        mask = (ci // BLK) <= (ri // BLK)

        def compute_group(g, wq16, wo16, q_scale, o_scale, is_first):
            kg = k_ref.at[g]
            vg = v_ref.at[g]
            q_raw = jnp.dot(x16, wq16,
                            preferred_element_type=jnp.float32)
            if q_scale is not None:
                q_raw = q_raw * q_scale
            q16 = q_raw.astype(jnp.bfloat16)
            head_cols = []
            for hh in range(HQ_PER):
                batch_rows = []
                for b in range(B_PER):
                    q = q16[b * SQ:(b + 1) * SQ, hh * DH:(hh + 1) * DH]
                    k = kg[hh, b]
                    v = vg[hh, b]
                    s = lax.dot_general(
                        q, k, (((1,), (1,)), ((), ())),
                        preferred_element_type=jnp.float32)
                    w = jnp.exp(jnp.where(mask, s, -1e9))
                    denom = jnp.sum(w, axis=1, keepdims=True)
                    ctx_h = jnp.dot(w.astype(jnp.bfloat16), v,
                                    preferred_element_type=jnp.float32)
                    batch_rows.append(ctx_h / denom)
                head_cols.append(jnp.concatenate(batch_rows, axis=0))
            ctx = jnp.concatenate(head_cols, axis=1)
            contrib = jnp.dot(ctx.astype(jnp.bfloat16), wo16,
                              preferred_element_type=jnp.float32)
            if o_scale is not None:
                contrib = contrib * o_scale
            return contrib

        if _KMODE == "comm":
            acc = jnp.zeros((B_PER * SQ, D_MODEL), jnp.float32)
        else:
            acc = compute_group(my_pos, wqv.astype(jnp.bfloat16),
                                wov.astype(jnp.bfloat16), None, None,
                                is_first=True)

        for d in (3, 1, 2):
            g = lax.rem(my_pos + d, N_DEV)
            if _KMODE != "compute":
                for comm, ssem, rsem in (
                    (sc_comm, sc_send, sc_recv),
                    (wq_comm, wq_send, wq_recv),
                    (wo_comm, wo_send, wo_recv),
                ):
                    recv = pltpu.make_async_remote_copy(
                        src_ref=comm.at[g],
                        dst_ref=comm.at[g],
                        send_sem=ssem.at[0],
                        recv_sem=rsem.at[g],
                        device_id=(g,),
                        device_id_type=pl.DeviceIdType.MESH,
                    )
                    recv.wait_recv()
            if _KMODE != "comm":
                if _KMODE == "compute":
                    acc = acc + compute_group(
                        g, wqv.astype(jnp.bfloat16),
                        wov.astype(jnp.bfloat16), None, None,
                        is_first=False)
                else:
                    sc = sc_comm[g]
                    acc = acc + compute_group(
                        g, wq_comm[g].astype(jnp.bfloat16),
                        wo_comm[g].astype(jnp.bfloat16),
                        sc[:, :D_GRP], sc[:, D_GRP:],
                        is_first=False)

        out_ref[...] = acc.reshape(B_PER, SQ, D_MODEL)

        for rdma in sends:
            rdma.wait_send()

    return pl.pallas_call(
        body,
        out_shape=jax.ShapeDtypeStruct((B_PER, SQ, D_MODEL), jnp.float32),
        in_specs=[pl.BlockSpec(memory_space=pltpu.VMEM)] * 5,
        out_specs=pl.BlockSpec(memory_space=pltpu.VMEM),
        scratch_shapes=[
            pltpu.VMEM((N_DEV, D_MODEL, D_GRP), jnp.int8),
            pltpu.VMEM((N_DEV, D_GRP, D_MODEL), jnp.int8),
            pltpu.VMEM((N_DEV, 1, D_GRP + D_MODEL), jnp.float32),
            pltpu.VMEM((D_MODEL, D_GRP), jnp.int8),
            pltpu.VMEM((D_GRP, D_MODEL), jnp.int8),
            pltpu.VMEM((1, D_GRP + D_MODEL), jnp.float32),
            pltpu.SemaphoreType.DMA((3,)),
            pltpu.SemaphoreType.DMA((N_DEV,)),
            pltpu.SemaphoreType.DMA((3,)),
            pltpu.SemaphoreType.DMA((N_DEV,)),
            pltpu.SemaphoreType.DMA((3,)),
            pltpu.SemaphoreType.DMA((N_DEV,)),
        ],
        compiler_params=pltpu.CompilerParams(
            collective_id=None if _KMODE == "compute" else 0),
    )(x, Wq, karr, varr, Wo)


# device time: 15557 ns/iter; 1.4473x vs baseline; 1.0317x over previous
import os

import jax
import jax.numpy as jnp
from jax import lax
from jax.experimental import pallas as pl
from jax.experimental.pallas import tpu as pltpu

_KMODE = "full"
_flag = os.path.join(os.path.dirname(os.path.abspath(__file__)), "kmode.txt")
if os.path.exists(_flag):
    _KMODE = open(_flag).read().strip() or "full"

N_DEV = 4
B_PER = 2
SQ = 128
HQ_PER = 4
DH = 64
D_MODEL = 512
D_GRP = HQ_PER * DH
BLK = 64


def kernel(x, Wq, K_ext, V_ext, Wo):
    my = lax.axis_index("i")
    k16 = lax.dynamic_slice_in_dim(K_ext, my * B_PER, B_PER, 0).astype(
        jnp.bfloat16)
    v16 = lax.dynamic_slice_in_dim(V_ext, my * B_PER, B_PER, 0).astype(
        jnp.bfloat16)
    karr = k16.transpose(2, 0, 1, 3).reshape(N_DEV, HQ_PER, B_PER, SQ, DH)
    varr = v16.transpose(2, 0, 1, 3).reshape(N_DEV, HQ_PER, B_PER, SQ, DH)

    def body(x_ref, wq_ref, k_ref, v_ref, wo_ref, out_ref,
             wq_comm, wo_comm, sc_comm, wq_stage, wo_stage, sc_stage,
             wq_send, wq_recv, wo_send, wo_recv, sc_send, sc_recv):
        my_pos = lax.axis_index("i")

        sends = []
        if _KMODE != "compute":
            barrier_sem = pltpu.get_barrier_semaphore()
            for d in (1, 2, 3):
                pl.semaphore_signal(
                    barrier_sem, inc=1,
                    device_id=(lax.rem(my_pos + d, N_DEV),),
                    device_id_type=pl.DeviceIdType.MESH,
                )

        wqv = wq_ref[...] * 0.125
        sq = jnp.max(jnp.abs(wqv), axis=0, keepdims=True) / 127.0
        wq_stage[...] = jnp.round(wqv / sq).astype(jnp.int8)
        wov = wo_ref[...]
        so = jnp.max(jnp.abs(wov), axis=0, keepdims=True) / 127.0
        wo_stage[...] = jnp.round(wov / so).astype(jnp.int8)
        sc_stage[:, :D_GRP] = sq
        sc_stage[:, D_GRP:] = so

        if _KMODE != "compute":
            pl.semaphore_wait(barrier_sem, 3)

            for d in (1, 3, 2):
                tgt = lax.rem(my_pos + d, N_DEV)
                for src, comm, ssem, rsem in (
                    (sc_stage, sc_comm, sc_send, sc_recv),
                    (wq_stage, wq_comm, wq_send, wq_recv),
                    (wo_stage, wo_comm, wo_send, wo_recv),
                ):
                    rdma = pltpu.make_async_remote_copy(
                        src_ref=src,
                        dst_ref=comm.at[my_pos],
                        send_sem=ssem.at[d - 1],
                        recv_sem=rsem.at[my_pos],
                        device_id=(tgt,),
                        device_id_type=pl.DeviceIdType.MESH,
                    )
                    rdma.start()
                    sends.append(rdma)

        x16 = x_ref[...].reshape(B_PER * SQ, D_MODEL).astype(jnp.bfloat16)
        ri = lax.broadcasted_iota(jnp.int32, (SQ, SQ), 0)
        ci = lax.broadcasted_iota(jnp.int32, (SQ, SQ), 1)
        mask = (ci // BLK) <= (ri // BLK)

        def compute_group(g, wq16, wo16, q_scale, o_scale, is_first):
            kg = k_ref.at[g]
            vg = v_ref.at[g]
            q_raw = jnp.dot(x16, wq16,
                            preferred_element_type=jnp.float32)
            if q_scale is not None:
                q_raw = q_raw * q_scale
            q16 = q_raw.astype(jnp.bfloat16)
            q_s = jnp.stack(
                [q16[b * SQ:(b + 1) * SQ, hh * DH:(hh + 1) * DH]
                 for hh in range(HQ_PER) for b in range(B_PER)],
                axis=0)
            k_s = kg[...].reshape(HQ_PER * B_PER, SQ, DH)
            v_s = vg[...].reshape(HQ_PER * B_PER, SQ, DH)
            s = lax.dot_general(
                q_s, k_s, (((2,), (2,)), ((0,), (0,))),
                preferred_element_type=jnp.float32)
            w = jnp.exp(jnp.where(mask[None], s, -1e9))
            denom = jnp.sum(w, axis=2, keepdims=True)
            ctx_s = lax.dot_general(
                w.astype(jnp.bfloat16), v_s, (((2,), (1,)), ((0,), (0,))),
                preferred_element_type=jnp.float32)
            ctx_s = ctx_s / denom
            ctx = jnp.concatenate(
                [jnp.concatenate([ctx_s[hh * B_PER + b] for b in range(B_PER)],
                                 axis=0)
                 for hh in range(HQ_PER)],
                axis=1)
            contrib = jnp.dot(ctx.astype(jnp.bfloat16), wo16,
                              preferred_element_type=jnp.float32)
            if o_scale is not None:
                contrib = contrib * o_scale
            return contrib

        if _KMODE == "comm":
            acc = jnp.zeros((B_PER * SQ, D_MODEL), jnp.float32)
        else:
            acc = compute_group(my_pos, wqv.astype(jnp.bfloat16),
                                wov.astype(jnp.bfloat16), None, None,
                                is_first=True)

        for d in (3, 1, 2):
            g = lax.rem(my_pos + d, N_DEV)
            if _KMODE != "compute":
                for comm, ssem, rsem in (
                    (sc_comm, sc_send, sc_recv),
                    (wq_comm, wq_send, wq_recv),
                    (wo_comm, wo_send, wo_recv),
                ):
                    recv = pltpu.make_async_remote_copy(
                        src_ref=comm.at[g],
                        dst_ref=comm.at[g],
                        send_sem=ssem.at[0],
                        recv_sem=rsem.at[g],
                        device_id=(g,),
                        device_id_type=pl.DeviceIdType.MESH,
                    )
                    recv.wait_recv()
            if _KMODE != "comm":
                if _KMODE == "compute":
                    acc = acc + compute_group(
                        g, wqv.astype(jnp.bfloat16),
                        wov.astype(jnp.bfloat16), None, None,
                        is_first=False)
                else:
                    sc = sc_comm[g]
                    acc = acc + compute_group(
                        g, wq_comm[g].astype(jnp.bfloat16),
                        wo_comm[g].astype(jnp.bfloat16),
                        sc[:, :D_GRP], sc[:, D_GRP:],
                        is_first=False)

        out_ref[...] = acc.reshape(B_PER, SQ, D_MODEL)

        for rdma in sends:
            rdma.wait_send()

    return pl.pallas_call(
        body,
        out_shape=jax.ShapeDtypeStruct((B_PER, SQ, D_MODEL), jnp.float32),
        in_specs=[pl.BlockSpec(memory_space=pltpu.VMEM)] * 5,
        out_specs=pl.BlockSpec(memory_space=pltpu.VMEM),
        scratch_shapes=[
            pltpu.VMEM((N_DEV, D_MODEL, D_GRP), jnp.int8),
            pltpu.VMEM((N_DEV, D_GRP, D_MODEL), jnp.int8),
            pltpu.VMEM((N_DEV, 1, D_GRP + D_MODEL), jnp.float32),
            pltpu.VMEM((D_MODEL, D_GRP), jnp.int8),
            pltpu.VMEM((D_GRP, D_MODEL), jnp.int8),
            pltpu.VMEM((1, D_GRP + D_MODEL), jnp.float32),
            pltpu.SemaphoreType.DMA((3,)),
            pltpu.SemaphoreType.DMA((N_DEV,)),
            pltpu.SemaphoreType.DMA((3,)),
            pltpu.SemaphoreType.DMA((N_DEV,)),
            pltpu.SemaphoreType.DMA((3,)),
            pltpu.SemaphoreType.DMA((N_DEV,)),
        ],
        compiler_params=pltpu.CompilerParams(
            collective_id=None if _KMODE == "compute" else 0),
    )(x, Wq, karr, varr, Wo)


# device time: 14839 ns/iter; 1.5173x vs baseline; 1.0484x over previous
import os

import jax
import jax.numpy as jnp
from jax import lax
from jax.experimental import pallas as pl
from jax.experimental.pallas import tpu as pltpu

_KMODE = "full"
_flag = os.path.join(os.path.dirname(os.path.abspath(__file__)), "kmode.txt")
if os.path.exists(_flag):
    _KMODE = open(_flag).read().strip() or "full"

N_DEV = 4
B_PER = 2
SQ = 128
HQ_PER = 4
DH = 64
D_MODEL = 512
D_GRP = HQ_PER * DH
BLK = 64


def kernel(x, Wq, K_ext, V_ext, Wo):
    my = lax.axis_index("i")
    k16 = lax.dynamic_slice_in_dim(K_ext, my * B_PER, B_PER, 0).astype(
        jnp.bfloat16)
    v16 = lax.dynamic_slice_in_dim(V_ext, my * B_PER, B_PER, 0).astype(
        jnp.bfloat16)
    karr = k16.transpose(2, 0, 1, 3).reshape(N_DEV, HQ_PER, B_PER, SQ, DH)
    varr = v16.transpose(2, 0, 1, 3).reshape(N_DEV, HQ_PER, B_PER, SQ, DH)

    def body(x_ref, wq_ref, k_ref, v_ref, wo_ref, out_ref,
             wq_comm, wo_comm, sc_comm, wq_stage, wo_stage, sc_stage,
             wq_send, wq_recv, wo_send, wo_recv, sc_send, sc_recv):
        my_pos = lax.axis_index("i")

        sends = []
        if _KMODE != "compute":
            barrier_sem = pltpu.get_barrier_semaphore()
            for d in (1, 2, 3):
                pl.semaphore_signal(
                    barrier_sem, inc=1,
                    device_id=(lax.rem(my_pos + d, N_DEV),),
                    device_id_type=pl.DeviceIdType.MESH,
                )

        wqv = wq_ref[...] * 0.125
        sq = jnp.max(jnp.abs(wqv), axis=0, keepdims=True) / 127.0
        wq_stage[...] = jnp.round(wqv / sq).astype(jnp.int8)
        wov = wo_ref[...]
        so = jnp.max(jnp.abs(wov), axis=0, keepdims=True) / 127.0
        wo_stage[...] = jnp.round(wov / so).astype(jnp.int8)
        sc_stage[:, :D_GRP] = sq
        sc_stage[:, D_GRP:] = so

        if _KMODE != "compute":
            pl.semaphore_wait(barrier_sem, 3)

            for d in (1, 3, 2):
                tgt = lax.rem(my_pos + d, N_DEV)
                for src, comm, ssem, rsem in (
                    (sc_stage, sc_comm, sc_send, sc_recv),
                    (wq_stage, wq_comm, wq_send, wq_recv),
                    (wo_stage, wo_comm, wo_send, wo_recv),
                ):
                    rdma = pltpu.make_async_remote_copy(
                        src_ref=src,
                        dst_ref=comm.at[my_pos],
                        send_sem=ssem.at[d - 1],
                        recv_sem=rsem.at[my_pos],
                        device_id=(tgt,),
                        device_id_type=pl.DeviceIdType.MESH,
                    )
                    rdma.start()
                    sends.append(rdma)

        x16 = x_ref[...].reshape(B_PER * SQ, D_MODEL).astype(jnp.bfloat16)
        ri = lax.broadcasted_iota(jnp.int32, (SQ, SQ), 0)
        ci = lax.broadcasted_iota(jnp.int32, (SQ, SQ), 1)
        mask = (ci // BLK) <= (ri // BLK)

        def compute_ctx(g, wq16, q_scale):
            kg = k_ref.at[g]
            vg = v_ref.at[g]
            q_raw = jnp.dot(x16, wq16,
                            preferred_element_type=jnp.float32)
            if q_scale is not None:
                q_raw = q_raw * q_scale
            q16 = q_raw.astype(jnp.bfloat16)
            q_s = jnp.stack(
                [q16[b * SQ:(b + 1) * SQ, hh * DH:(hh + 1) * DH]
                 for hh in range(HQ_PER) for b in range(B_PER)],
                axis=0)
            k_s = kg[...].reshape(HQ_PER * B_PER, SQ, DH)
            v_s = vg[...].reshape(HQ_PER * B_PER, SQ, DH)
            s = lax.dot_general(
                q_s, k_s, (((2,), (2,)), ((0,), (0,))),
                preferred_element_type=jnp.float32)
            w = jnp.exp(jnp.where(mask[None], s, -1e9))
            denom = jnp.sum(w, axis=2, keepdims=True)
            ctx_s = lax.dot_general(
                w.astype(jnp.bfloat16), v_s, (((2,), (1,)), ((0,), (0,))),
                preferred_element_type=jnp.float32)
            ctx_s = ctx_s / denom
            ctx = jnp.concatenate(
                [jnp.concatenate([ctx_s[hh * B_PER + b] for b in range(B_PER)],
                                 axis=0)
                 for hh in range(HQ_PER)],
                axis=1)
            return ctx.astype(jnp.bfloat16)

        def project_out(ctx16, wo16, o_scale):
            contrib = jnp.dot(ctx16, wo16,
                              preferred_element_type=jnp.float32)
            if o_scale is not None:
                contrib = contrib * o_scale
            return contrib

        if _KMODE == "comm":
            acc = jnp.zeros((B_PER * SQ, D_MODEL), jnp.float32)
        else:
            acc = project_out(compute_ctx(my_pos, wqv.astype(jnp.bfloat16),
                                          None),
                              wov.astype(jnp.bfloat16), None)

        def wait_for(comm, ssem, rsem, g):
            recv = pltpu.make_async_remote_copy(
                src_ref=comm.at[g],
                dst_ref=comm.at[g],
                send_sem=ssem.at[0],
                recv_sem=rsem.at[g],
                device_id=(g,),
                device_id_type=pl.DeviceIdType.MESH,
            )
            recv.wait_recv()

        for d in (3, 1, 2):
            g = lax.rem(my_pos + d, N_DEV)
            if _KMODE == "compute":
                acc = acc + project_out(
                    compute_ctx(g, wqv.astype(jnp.bfloat16), None),
                    wov.astype(jnp.bfloat16), None)
            elif _KMODE == "comm":
                wait_for(sc_comm, sc_send, sc_recv, g)
                wait_for(wq_comm, wq_send, wq_recv, g)
                wait_for(wo_comm, wo_send, wo_recv, g)
            else:
                wait_for(sc_comm, sc_send, sc_recv, g)
                wait_for(wq_comm, wq_send, wq_recv, g)
                sc = sc_comm[g]
                ctx16 = compute_ctx(g, wq_comm[g].astype(jnp.bfloat16),
                                    sc[:, :D_GRP])
                wait_for(wo_comm, wo_send, wo_recv, g)
                acc = acc + project_out(
                    ctx16, wo_comm[g].astype(jnp.bfloat16), sc[:, D_GRP:])

        out_ref[...] = acc.reshape(B_PER, SQ, D_MODEL)

        for rdma in sends:
            rdma.wait_send()

    return pl.pallas_call(
        body,
        out_shape=jax.ShapeDtypeStruct((B_PER, SQ, D_MODEL), jnp.float32),
        in_specs=[pl.BlockSpec(memory_space=pltpu.VMEM)] * 5,
        out_specs=pl.BlockSpec(memory_space=pltpu.VMEM),
        scratch_shapes=[
            pltpu.VMEM((N_DEV, D_MODEL, D_GRP), jnp.int8),
            pltpu.VMEM((N_DEV, D_GRP, D_MODEL), jnp.int8),
            pltpu.VMEM((N_DEV, 1, D_GRP + D_MODEL), jnp.float32),
            pltpu.VMEM((D_MODEL, D_GRP), jnp.int8),
            pltpu.VMEM((D_GRP, D_MODEL), jnp.int8),
            pltpu.VMEM((1, D_GRP + D_MODEL), jnp.float32),
            pltpu.SemaphoreType.DMA((3,)),
            pltpu.SemaphoreType.DMA((N_DEV,)),
            pltpu.SemaphoreType.DMA((3,)),
            pltpu.SemaphoreType.DMA((N_DEV,)),
            pltpu.SemaphoreType.DMA((3,)),
            pltpu.SemaphoreType.DMA((N_DEV,)),
        ],
        compiler_params=pltpu.CompilerParams(
            collective_id=None if _KMODE == "compute" else 0),
    )(x, Wq, karr, varr, Wo)
